# baseline (speedup 1.0000x reference)
import functools

import jax
import jax.numpy as jnp
from jax import lax
from jax.experimental import pallas as pl
from jax.experimental.pallas import tpu as pltpu
from jax.experimental.pallas import tpu_sc as plsc

B, S, D, A, E = 2, 2048, 1024, 128, 8
T = B * S
T2 = 2 * T
BT = 512
BTS = 256
MAXB = T2 // BTS + E
PMAX = MAXB * BTS
NC, NS = 2, 16
NW = NC * NS
ROWS_K3 = PMAX // NW
ROWS_K5 = T2 // NW
CH = 32
DP = D // 2


def _pack(lo, hi):
    lo_b = lax.shift_right_logical(
        lax.bitcast_convert_type(lo.astype(jnp.bfloat16).astype(jnp.float32),
                                 jnp.int32), 16)
    hi_b = lax.bitcast_convert_type(
        hi.astype(jnp.bfloat16).astype(jnp.float32), jnp.int32) & (-65536)
    return hi_b | lo_b


def _unpack(pk):
    lo = lax.bitcast_convert_type(lax.shift_left(pk, 16), jnp.float32)
    hi = lax.bitcast_convert_type(pk & (-65536), jnp.float32)
    return lo, hi


def _gate_kernel(a_ref, wg_ref, bg_ref, h_ref,
                 ii_ref, gg_ref, pwe_ref, cnt_ref, bexp_ref, val_ref,
                 hpk_ref, ent_ref, basev):
    i = pl.program_id(0)
    n_blocks = T // BT

    @pl.when(i == 0)
    def _():
        basev[...] = jnp.zeros((1, E), jnp.float32)
        ent_ref[0, 0] = jnp.float32(0.0)

    h = h_ref[...]
    h_bf = h.astype(jnp.bfloat16)
    hpk_ref[...] = _pack(h[:, :DP], h[:, DP:])

    wg = wg_ref[...]
    wg_h = wg[:, :D].astype(jnp.bfloat16)
    wg_a = wg[:, D:].astype(jnp.bfloat16)
    b_idx = i * BT // S
    a_bf = a_ref[pl.ds(b_idx, 1), :].astype(jnp.bfloat16)
    logits = jax.lax.dot_general(
        h_bf, wg_h, (((1,), (1,)), ((), ())),
        preferred_element_type=jnp.float32)
    logits_a = jax.lax.dot_general(
        a_bf, wg_a, (((1,), (1,)), ((), ())),
        preferred_element_type=jnp.float32)
    logits = logits + logits_a + bg_ref[...]

    m = jnp.max(logits, axis=-1, keepdims=True)
    p = jnp.exp(logits - m)
    s = jnp.sum(p, axis=-1, keepdims=True)
    g = p / s

    iota = lax.broadcasted_iota(jnp.int32, (BT, E), 1)
    m1 = jnp.max(g, axis=-1, keepdims=True)
    i1 = jnp.min(jnp.where(g == m1, iota, E), axis=-1, keepdims=True)
    gm = jnp.where(iota == i1, -jnp.inf, g)
    m2 = jnp.max(gm, axis=-1, keepdims=True)
    i2 = jnp.min(jnp.where(gm == m2, iota, E), axis=-1, keepdims=True)
    den = m1 + m2
    g1n = m1 / den
    g2n = m2 / den

    ent_ref[0, 0] += jnp.sum(g * jnp.log(g + 1e-10))

    m1hot = (iota == i1).astype(jnp.bfloat16)
    m2hot = (iota == i2).astype(jnp.bfloat16)
    r = lax.broadcasted_iota(jnp.int32, (BT, BT), 0)
    c = lax.broadcasted_iota(jnp.int32, (BT, BT), 1)
    ltri = (c < r).astype(jnp.bfloat16)
    cex = jax.lax.dot_general(
        ltri, m1hot + m2hot, (((1,), (0,)), ((), ())),
        preferred_element_type=jnp.float32)
    pvec = basev[...] + cex
    pwe1 = jnp.sum(jnp.where(iota == i1, pvec, 0.0), axis=-1, keepdims=True)
    pwe2 = jnp.sum(jnp.where(iota == i2, pvec, 0.0), axis=-1, keepdims=True)
    basev[...] += jnp.sum(m1hot + m2hot, axis=0, keepdims=True
                          ).astype(jnp.float32)

    ii_ref[...] = jnp.concatenate([i1, i2], axis=1)
    gg_ref[...] = jnp.concatenate([g1n, g2n], axis=1)
    pwe_ref[...] = jnp.concatenate([pwe1, pwe2], axis=1).astype(jnp.int32)

    @pl.when(i == n_blocks - 1)
    def _():
        ent_ref[0, 0] = ent_ref[0, 0] * jnp.float32(-1.0 / T)
        cnt = basev[...]
        lane8 = lax.broadcasted_iota(jnp.int32, (1, E), 1)
        iota64 = lax.broadcasted_iota(jnp.int32, (1, 64), 1
                                      ).astype(jnp.float32)
        bexp = jnp.full((1, 64), -1.0, jnp.float32)
        val = jnp.zeros((1, 64), jnp.float32)
        pbase = jnp.zeros((1, E), jnp.float32)
        run = jnp.float32(0.0)
        for e in range(E):
            ce = jnp.sum(jnp.where(lane8 == e, cnt, 0.0))
            cpad = jnp.floor((ce + (BTS - 1)) / BTS) * BTS
            pbase = pbase + jnp.where(lane8 == e, run, 0.0)
            blkbase = run / BTS
            bexp = bexp + (iota64 >= blkbase).astype(jnp.float32)
            val = val + jnp.where(
                bexp == e,
                jnp.clip(ce - (iota64 - blkbase) * BTS, 0.0, float(BTS)),
                0.0)
            run = run + cpad
        pbase16 = jnp.concatenate(
            [pbase, jnp.zeros((1, 8), jnp.float32)], axis=1)
        cnt_ref[...] = pbase16.astype(jnp.int32)
        bexp_ref[...] = jnp.clip(bexp, 0.0, float(E - 1)).astype(jnp.int32)
        val_ref[...] = val.astype(jnp.int32)


def _pos_kernel(ii_ref, pwe_ref, pbase_ref, sp_ref):
    lane16 = lax.broadcasted_iota(jnp.int32, (1, 16), 1)
    ii = ii_ref[...]
    acc = jnp.zeros((T, 2), jnp.int32)
    pbase = pbase_ref[...]
    for e in range(E):
        be_s = jnp.sum(jnp.where(lane16 == e, pbase, 0))
        acc = acc + jnp.where(ii == e, be_s, 0)
    sp_ref[...] = acc + pwe_ref[...]


def _scatter_body(sp_hbm, tok_hbm, gg_hbm, stok_hbm, sgat_hbm,
                  sp_v, tok_v, gg_v, sem_a, sem_b):
    cid = lax.axis_index("c")
    sid = lax.axis_index("s")
    wid = sid * NC + cid
    j0 = wid * 2
    pltpu.sync_copy(sp_hbm.at[pl.ds(j0, 2)], sp_v)
    pltpu.sync_copy(tok_hbm.at[pl.ds(j0, 2)], tok_v)
    pltpu.sync_copy(gg_hbm.at[pl.ds(j0, 2)], gg_v)
    handles = []
    for r in range(2):
        handles.append(pltpu.async_copy(
            tok_v.at[r], stok_hbm.at[sp_v.at[r]], sem_a))
        handles.append(pltpu.async_copy(
            gg_v.at[r], sgat_hbm.at[sp_v.at[r]], sem_b))
    for hd in handles:
        hd.wait()


def _gather_body(idx_hbm, tab_hbm, out_hbm, idx_v, bufs, sem,
                 *, rows_per_w, clamp_hi, depth):
    cid = lax.axis_index("c")
    sid = lax.axis_index("s")
    wid = sid * NC + cid
    base = wid * rows_per_w
    nch = rows_per_w // CH

    pltpu.sync_copy(idx_hbm.at[pl.ds(base, rows_per_w)], idx_v)
    for k in range(rows_per_w // 16):
        sl = pl.ds(k * 16, 16)
        idx_v[sl] = jnp.clip(idx_v[sl], 0, clamp_hi)

    def fire(ch):
        return pltpu.async_copy(
            tab_hbm.at[idx_v.at[pl.ds(ch * CH, CH)]],
            bufs.at[ch % depth], sem.at[ch % depth])

    live = [fire(ch) for ch in range(min(depth, nch))]
    for ch in range(nch):
        live[ch % depth].wait()
        pltpu.sync_copy(bufs.at[ch % depth],
                        out_hbm.at[pl.ds(base + ch * CH, CH)])
        if ch + depth < nch:
            live[ch % depth] = fire(ch + depth)


def _expert_kernel(bexp_ref, val_ref, hg_ref, sg_ref, we_ref, be_ref, y_ref):
    i = pl.program_id(0)
    v = val_ref[i]
    e = bexp_ref[i]
    g = sg_ref[...]
    rio = lax.broadcasted_iota(jnp.int32, (BTS, 1), 0)
    gv = jnp.where(rio < v, g, 0.0)
    h_lo, h_hi = _unpack(hg_ref[...])
    hf = jnp.concatenate([h_lo, h_hi], axis=1)
    hb = (hf * gv).astype(jnp.bfloat16)
    wb = we_ref[0]
    y = jax.lax.dot_general(
        hb, wb, (((1,), (1,)), ((), ())),
        preferred_element_type=jnp.float32)
    y = y + gv * be_ref[pl.ds(e, 1), :]
    y_ref[...] = _pack(y[:, :DP], y[:, DP:])


def _pairsum_kernel(ys_ref, out_ref):
    ys = ys_ref[...]
    a_lo, a_hi = _unpack(ys[:, :DP])
    b_lo, b_hi = _unpack(ys[:, DP:])
    out_ref[...] = jnp.concatenate(
        [a_lo + b_lo, a_hi + b_hi], axis=1)


@jax.jit
def kernel(h, a, We, be, Wg, bg):
    h2 = h.reshape(T, D)
    bg2 = bg.reshape(1, E)

    n_blocks = T // BT
    ii, gg, pwe, cnt16, bexp64, val64, hpk, ent = pl.pallas_call(
        _gate_kernel,
        grid=(n_blocks,),
        in_specs=[
            pl.BlockSpec((B, A), lambda i: (0, 0)),
            pl.BlockSpec((E, D + A), lambda i: (0, 0)),
            pl.BlockSpec((1, E), lambda i: (0, 0)),
            pl.BlockSpec((BT, D), lambda i: (i, 0)),
        ],
        out_specs=[
            pl.BlockSpec((BT, 2), lambda i: (i, 0)),
            pl.BlockSpec((BT, 2), lambda i: (i, 0)),
            pl.BlockSpec((BT, 2), lambda i: (i, 0)),
            pl.BlockSpec((1, 16), lambda i: (0, 0)),
            pl.BlockSpec((1, 64), lambda i: (0, 0)),
            pl.BlockSpec((1, 64), lambda i: (0, 0)),
            pl.BlockSpec((BT, DP), lambda i: (i, 0)),
            pl.BlockSpec(memory_space=pltpu.SMEM),
        ],
        out_shape=[
            jax.ShapeDtypeStruct((T, 2), jnp.int32),
            jax.ShapeDtypeStruct((T, 2), jnp.float32),
            jax.ShapeDtypeStruct((T, 2), jnp.int32),
            jax.ShapeDtypeStruct((1, 16), jnp.int32),
            jax.ShapeDtypeStruct((1, 64), jnp.int32),
            jax.ShapeDtypeStruct((1, 64), jnp.int32),
            jax.ShapeDtypeStruct((T, DP), jnp.int32),
            jax.ShapeDtypeStruct((1, 1), jnp.float32),
        ],
        scratch_shapes=[pltpu.VMEM((1, E), jnp.float32)],
        compiler_params=pltpu.CompilerParams(
            dimension_semantics=("arbitrary",)),
    )(a, Wg, bg2, h2)

    mesh = plsc.VectorSubcoreMesh(core_axis_name="c", subcore_axis_name="s")

    sp = pl.pallas_call(
        _pos_kernel,
        in_specs=[
            pl.BlockSpec((T, 2), lambda: (0, 0)),
            pl.BlockSpec((T, 2), lambda: (0, 0)),
            pl.BlockSpec((1, 16), lambda: (0, 0)),
        ],
        out_specs=pl.BlockSpec((T, 2), lambda: (0, 0)),
        out_shape=jax.ShapeDtypeStruct((T, 2), jnp.int32),
    )(ii, pwe, cnt16)
    pos = sp.reshape(T2)

    toks = (jnp.arange(T2, dtype=jnp.int32) // 2).reshape(64, 128)
    stok, sgat = pl.kernel(
        _scatter_body,
        out_type=[
            jax.ShapeDtypeStruct((PMAX,), jnp.int32),
            jax.ShapeDtypeStruct((PMAX,), jnp.float32),
        ],
        mesh=mesh,
        scratch_types=[
            pltpu.VMEM((2, 128), jnp.int32),
            pltpu.VMEM((2, 128), jnp.int32),
            pltpu.VMEM((2, 128), jnp.float32),
            pltpu.SemaphoreType.DMA,
            pltpu.SemaphoreType.DMA,
        ],
    )(sp.reshape(64, 128), toks, gg.reshape(64, 128))

    hg = pl.kernel(
        functools.partial(_gather_body, rows_per_w=ROWS_K3, clamp_hi=T - 1,
                          depth=4),
        out_type=jax.ShapeDtypeStruct((PMAX, DP), jnp.int32),
        mesh=mesh,
        scratch_types=[
            pltpu.VMEM((ROWS_K3,), jnp.int32),
            pltpu.VMEM((4, CH, DP), jnp.int32),
            pltpu.SemaphoreType.DMA((4,)),
        ],
    )(stok, hpk)

    y = pl.pallas_call(
        _expert_kernel,
        grid_spec=pltpu.PrefetchScalarGridSpec(
            num_scalar_prefetch=2,
            grid=(MAXB,),
            in_specs=[
                pl.BlockSpec((BTS, DP), lambda i, be_, va_: (i, 0)),
                pl.BlockSpec((BTS, 1), lambda i, be_, va_: (i, 0)),
                pl.BlockSpec((1, D, D), lambda i, be_, va_: (be_[i], 0, 0)),
                pl.BlockSpec((E, D), lambda i, be_, va_: (0, 0)),
            ],
            out_specs=pl.BlockSpec((BTS, DP), lambda i, be_, va_: (i, 0)),
        ),
        out_shape=jax.ShapeDtypeStruct((PMAX, DP), jnp.int32),
        compiler_params=pltpu.CompilerParams(
            dimension_semantics=("arbitrary",)),
    )(bexp64.reshape(64)[:MAXB], val64.reshape(64)[:MAXB],
      hg, sgat.reshape(PMAX, 1), We.astype(jnp.bfloat16), be)

    yg = pl.kernel(
        functools.partial(_gather_body, rows_per_w=ROWS_K5,
                          clamp_hi=PMAX - 1, depth=4),
        out_type=jax.ShapeDtypeStruct((T2, DP), jnp.int32),
        mesh=mesh,
        scratch_types=[
            pltpu.VMEM((ROWS_K5,), jnp.int32),
            pltpu.VMEM((4, CH, DP), jnp.int32),
            pltpu.SemaphoreType.DMA((4,)),
        ],
    )(pos, y)

    out = pl.pallas_call(
        _pairsum_kernel,
        grid=(n_blocks,),
        in_specs=[pl.BlockSpec((BT, 2 * DP), lambda i: (i, 0))],
        out_specs=pl.BlockSpec((BT, D), lambda i: (i, 0)),
        out_shape=jax.ShapeDtypeStruct((T, D), jnp.float32),
    )(yg.reshape(T, 2 * DP))

    return (out.reshape(B, S, D), ent[0, 0], jnp.float32(0.0))


# speedup vs baseline: 1.5573x; 1.5573x over previous
import functools

import jax
import jax.numpy as jnp
from jax import lax
from jax.experimental import pallas as pl
from jax.experimental.pallas import tpu as pltpu
from jax.experimental.pallas import tpu_sc as plsc

B, S, D, A, E = 2, 2048, 1024, 128, 8
T = B * S
T2 = 2 * T
BT = 512
BTS = 256
MAXB = T2 // BTS + E
PMAX = MAXB * BTS
NC, NS = 2, 16
NW = NC * NS
ROWS_K3 = PMAX // NW
ROWS_K5 = T2 // NW
CH = 32
DP = D // 2


def _pack(lo, hi):
    lo_b = lax.shift_right_logical(
        lax.bitcast_convert_type(lo.astype(jnp.bfloat16).astype(jnp.float32),
                                 jnp.int32), 16)
    hi_b = lax.bitcast_convert_type(
        hi.astype(jnp.bfloat16).astype(jnp.float32), jnp.int32) & (-65536)
    return hi_b | lo_b


def _unpack(pk):
    lo = lax.bitcast_convert_type(lax.shift_left(pk, 16), jnp.float32)
    hi = lax.bitcast_convert_type(pk & (-65536), jnp.float32)
    return lo, hi


def _gate_kernel(a_ref, wg_ref, bg_ref, h_ref,
                 ii_ref, gg_ref, pwe_ref, cnt_ref, bexp_ref, val_ref,
                 hp1_ref, hp2_ref, ent_ref, basev):
    i = pl.program_id(0)
    n_blocks = T // BT

    @pl.when(i == 0)
    def _():
        basev[...] = jnp.zeros((1, E), jnp.float32)
        ent_ref[0, 0] = jnp.float32(0.0)

    h = h_ref[...]
    h_bf = h.astype(jnp.bfloat16)

    wg = wg_ref[...]
    wg_h = wg[:, :D].astype(jnp.bfloat16)
    wg_a = wg[:, D:].astype(jnp.bfloat16)
    b_idx = i * BT // S
    a_bf = a_ref[pl.ds(b_idx, 1), :].astype(jnp.bfloat16)
    logits = jax.lax.dot_general(
        h_bf, wg_h, (((1,), (1,)), ((), ())),
        preferred_element_type=jnp.float32)
    logits_a = jax.lax.dot_general(
        a_bf, wg_a, (((1,), (1,)), ((), ())),
        preferred_element_type=jnp.float32)
    logits = logits + logits_a + bg_ref[...]

    m = jnp.max(logits, axis=-1, keepdims=True)
    p = jnp.exp(logits - m)
    s = jnp.sum(p, axis=-1, keepdims=True)
    g = p / s

    iota = lax.broadcasted_iota(jnp.int32, (BT, E), 1)
    m1 = jnp.max(g, axis=-1, keepdims=True)
    i1 = jnp.min(jnp.where(g == m1, iota, E), axis=-1, keepdims=True)
    gm = jnp.where(iota == i1, -jnp.inf, g)
    m2 = jnp.max(gm, axis=-1, keepdims=True)
    i2 = jnp.min(jnp.where(gm == m2, iota, E), axis=-1, keepdims=True)
    den = m1 + m2
    g1n = m1 / den
    g2n = m2 / den

    ent_ref[0, 0] += jnp.sum(g * jnp.log(g + 1e-10))

    m1hot = (iota == i1).astype(jnp.bfloat16)
    m2hot = (iota == i2).astype(jnp.bfloat16)
    r = lax.broadcasted_iota(jnp.int32, (BT, BT), 0)
    c = lax.broadcasted_iota(jnp.int32, (BT, BT), 1)
    ltri = (c < r).astype(jnp.bfloat16)
    cex = jax.lax.dot_general(
        ltri, m1hot + m2hot, (((1,), (0,)), ((), ())),
        preferred_element_type=jnp.float32)
    pvec = basev[...] + cex
    pwe1 = jnp.sum(jnp.where(iota == i1, pvec, 0.0), axis=-1, keepdims=True)
    pwe2 = jnp.sum(jnp.where(iota == i2, pvec, 0.0), axis=-1, keepdims=True)
    basev[...] += jnp.sum(m1hot + m2hot, axis=0, keepdims=True
                          ).astype(jnp.float32)

    ii_ref[...] = jnp.concatenate([i1, i2], axis=1)
    gg_ref[...] = jnp.concatenate([g1n, g2n], axis=1)
    pwe_ref[...] = jnp.concatenate([pwe1, pwe2], axis=1).astype(jnp.int32)
    hp1_ref[...] = _pack(h[:, :DP] * g1n, h[:, DP:] * g1n)
    hp2_ref[...] = _pack(h[:, :DP] * g2n, h[:, DP:] * g2n)

    @pl.when(i == n_blocks - 1)
    def _():
        ent_ref[0, 0] = ent_ref[0, 0] * jnp.float32(-1.0 / T)
        cnt = basev[...]
        lane8 = lax.broadcasted_iota(jnp.int32, (1, E), 1)
        iota64 = lax.broadcasted_iota(jnp.int32, (1, 64), 1
                                      ).astype(jnp.float32)
        bexp = jnp.full((1, 64), -1.0, jnp.float32)
        val = jnp.zeros((1, 64), jnp.float32)
        pbase = jnp.zeros((1, E), jnp.float32)
        run = jnp.float32(0.0)
        for e in range(E):
            ce = jnp.sum(jnp.where(lane8 == e, cnt, 0.0))
            cpad = jnp.floor((ce + (BTS - 1)) / BTS) * BTS
            pbase = pbase + jnp.where(lane8 == e, run, 0.0)
            blkbase = run / BTS
            bexp = bexp + (iota64 >= blkbase).astype(jnp.float32)
            val = val + jnp.where(
                bexp == e,
                jnp.clip(ce - (iota64 - blkbase) * BTS, 0.0, float(BTS)),
                0.0)
            run = run + cpad
        pbase16 = jnp.concatenate(
            [pbase, jnp.zeros((1, 8), jnp.float32)], axis=1)
        cnt_ref[...] = pbase16.astype(jnp.int32)
        bexp_ref[...] = jnp.clip(bexp, 0.0, float(E - 1)).astype(jnp.int32)
        val_ref[...] = val.astype(jnp.int32)


def _pos_kernel(ii_ref, pwe_ref, pbase_ref, sp_ref, s1_ref, s2_ref):
    lane16 = lax.broadcasted_iota(jnp.int32, (1, 16), 1)
    ii = ii_ref[...]
    acc = jnp.zeros((T, 2), jnp.int32)
    pbase = pbase_ref[...]
    for e in range(E):
        be_s = jnp.sum(jnp.where(lane16 == e, pbase, 0))
        acc = acc + jnp.where(ii == e, be_s, 0)
    sp = acc + pwe_ref[...]
    sp_ref[...] = sp
    s1_ref[...] = sp[:, 0:1]
    s2_ref[...] = sp[:, 1:2]


def _rowscatter_body(s1_hbm, s2_hbm, h1_hbm, h2_hbm, hs_hbm,
                     s1_v, s2_v, bufs1, bufs2, sem1, sem2, sem_a, sem_b):
    cid = lax.axis_index("c")
    sid = lax.axis_index("s")
    wid = sid * NC + cid
    t0 = wid * (T // NW)
    pltpu.sync_copy(s1_hbm.at[pl.ds(t0, T // NW)], s1_v)
    pltpu.sync_copy(s2_hbm.at[pl.ds(t0, T // NW)], s2_v)
    SCH = 16
    nch = (T // NW) // SCH

    def fire(c):
        return (
            pltpu.async_copy(h1_hbm.at[pl.ds(t0 + c * SCH, SCH)],
                             bufs1.at[c % 2], sem1.at[c % 2]),
            pltpu.async_copy(h2_hbm.at[pl.ds(t0 + c * SCH, SCH)],
                             bufs2.at[c % 2], sem2.at[c % 2]))

    live = fire(0)
    for c in range(nch):
        nxt = fire(c + 1) if c + 1 < nch else None
        live[0].wait()
        live[1].wait()
        sl = pl.ds(c * SCH, SCH)
        a = pltpu.async_copy(bufs1.at[c % 2], hs_hbm.at[s1_v.at[sl]], sem_a)
        b = pltpu.async_copy(bufs2.at[c % 2], hs_hbm.at[s2_v.at[sl]], sem_b)
        a.wait()
        b.wait()
        live = nxt


def _gather_body(idx_hbm, tab_hbm, out_hbm, idx_v, bufs, sem,
                 *, rows_per_w, clamp_hi, depth):
    cid = lax.axis_index("c")
    sid = lax.axis_index("s")
    wid = sid * NC + cid
    base = wid * rows_per_w
    nch = rows_per_w // CH

    pltpu.sync_copy(idx_hbm.at[pl.ds(base, rows_per_w)], idx_v)
    for k in range(rows_per_w // 16):
        sl = pl.ds(k * 16, 16)
        idx_v[sl] = jnp.clip(idx_v[sl], 0, clamp_hi)

    def fire(ch):
        return pltpu.async_copy(
            tab_hbm.at[idx_v.at[pl.ds(ch * CH, CH)]],
            bufs.at[ch % depth], sem.at[ch % depth])

    live = [fire(ch) for ch in range(min(depth, nch))]
    for ch in range(nch):
        live[ch % depth].wait()
        pltpu.sync_copy(bufs.at[ch % depth],
                        out_hbm.at[pl.ds(base + ch * CH, CH)])
        if ch + depth < nch:
            live[ch % depth] = fire(ch + depth)


def _expert_kernel(bexp_ref, hg_ref, we_ref, y_ref):
    h_lo, h_hi = _unpack(hg_ref[...])
    hf = jnp.concatenate([h_lo, h_hi], axis=1)
    hb = hf.astype(jnp.bfloat16)
    wb = we_ref[0]
    y = jax.lax.dot_general(
        hb, wb, (((1,), (1,)), ((), ())),
        preferred_element_type=jnp.float32)
    y_ref[...] = _pack(y[:, :DP], y[:, DP:])


def _pairsum_kernel(ys_ref, ii_ref, gg_ref, be_ref, out_ref):
    ys = ys_ref[...]
    a_lo, a_hi = _unpack(ys[:, :DP])
    b_lo, b_hi = _unpack(ys[:, DP:])
    lo = a_lo + b_lo
    hi = a_hi + b_hi
    ii = ii_ref[...]
    gg = gg_ref[...]
    i1 = ii[:, 0:1]
    i2 = ii[:, 1:2]
    g1 = gg[:, 0:1]
    g2 = gg[:, 1:2]
    for e in range(E):
        w = jnp.where(i1 == e, g1, 0.0) + jnp.where(i2 == e, g2, 0.0)
        bev = be_ref[pl.ds(e, 1), :]
        lo = lo + w * bev[:, :DP]
        hi = hi + w * bev[:, DP:]
    out_ref[...] = jnp.concatenate([lo, hi], axis=1)


@jax.jit
def kernel(h, a, We, be, Wg, bg):
    h2 = h.reshape(T, D)
    bg2 = bg.reshape(1, E)

    n_blocks = T // BT
    ii, gg, pwe, cnt16, bexp64, val64, hp1, hp2, ent = pl.pallas_call(
        _gate_kernel,
        grid=(n_blocks,),
        in_specs=[
            pl.BlockSpec((B, A), lambda i: (0, 0)),
            pl.BlockSpec((E, D + A), lambda i: (0, 0)),
            pl.BlockSpec((1, E), lambda i: (0, 0)),
            pl.BlockSpec((BT, D), lambda i: (i, 0)),
        ],
        out_specs=[
            pl.BlockSpec((BT, 2), lambda i: (i, 0)),
            pl.BlockSpec((BT, 2), lambda i: (i, 0)),
            pl.BlockSpec((BT, 2), lambda i: (i, 0)),
            pl.BlockSpec((1, 16), lambda i: (0, 0)),
            pl.BlockSpec((1, 64), lambda i: (0, 0)),
            pl.BlockSpec((1, 64), lambda i: (0, 0)),
            pl.BlockSpec((BT, DP), lambda i: (i, 0)),
            pl.BlockSpec((BT, DP), lambda i: (i, 0)),
            pl.BlockSpec(memory_space=pltpu.SMEM),
        ],
        out_shape=[
            jax.ShapeDtypeStruct((T, 2), jnp.int32),
            jax.ShapeDtypeStruct((T, 2), jnp.float32),
            jax.ShapeDtypeStruct((T, 2), jnp.int32),
            jax.ShapeDtypeStruct((1, 16), jnp.int32),
            jax.ShapeDtypeStruct((1, 64), jnp.int32),
            jax.ShapeDtypeStruct((1, 64), jnp.int32),
            jax.ShapeDtypeStruct((T, DP), jnp.int32),
            jax.ShapeDtypeStruct((T, DP), jnp.int32),
            jax.ShapeDtypeStruct((1, 1), jnp.float32),
        ],
        scratch_shapes=[pltpu.VMEM((1, E), jnp.float32)],
        compiler_params=pltpu.CompilerParams(
            dimension_semantics=("arbitrary",)),
    )(a, Wg, bg2, h2)

    mesh = plsc.VectorSubcoreMesh(core_axis_name="c", subcore_axis_name="s")

    sp, sp1, sp2 = pl.pallas_call(
        _pos_kernel,
        in_specs=[
            pl.BlockSpec((T, 2), lambda: (0, 0)),
            pl.BlockSpec((T, 2), lambda: (0, 0)),
            pl.BlockSpec((1, 16), lambda: (0, 0)),
        ],
        out_specs=[
            pl.BlockSpec((T, 2), lambda: (0, 0)),
            pl.BlockSpec((T, 1), lambda: (0, 0)),
            pl.BlockSpec((T, 1), lambda: (0, 0)),
        ],
        out_shape=[
            jax.ShapeDtypeStruct((T, 2), jnp.int32),
            jax.ShapeDtypeStruct((T, 1), jnp.int32),
            jax.ShapeDtypeStruct((T, 1), jnp.int32),
        ],
    )(ii, pwe, cnt16)
    pos = sp.reshape(T2)

    hg = pl.kernel(
        _rowscatter_body,
        out_type=jax.ShapeDtypeStruct((PMAX, DP), jnp.int32),
        mesh=mesh,
        scratch_types=[
            pltpu.VMEM((T // NW,), jnp.int32),
            pltpu.VMEM((T // NW,), jnp.int32),
            pltpu.VMEM((2, 16, DP), jnp.int32),
            pltpu.VMEM((2, 16, DP), jnp.int32),
            pltpu.SemaphoreType.DMA((2,)),
            pltpu.SemaphoreType.DMA((2,)),
            pltpu.SemaphoreType.DMA,
            pltpu.SemaphoreType.DMA,
        ],
    )(sp1.reshape(T), sp2.reshape(T), hp1, hp2)

    y = pl.pallas_call(
        _expert_kernel,
        grid_spec=pltpu.PrefetchScalarGridSpec(
            num_scalar_prefetch=1,
            grid=(MAXB,),
            in_specs=[
                pl.BlockSpec((BTS, DP), lambda i, be_: (i, 0)),
                pl.BlockSpec((1, D, D), lambda i, be_: (be_[i], 0, 0)),
            ],
            out_specs=pl.BlockSpec((BTS, DP), lambda i, be_: (i, 0)),
        ),
        out_shape=jax.ShapeDtypeStruct((PMAX, DP), jnp.int32),
        compiler_params=pltpu.CompilerParams(
            dimension_semantics=("arbitrary",)),
    )(bexp64.reshape(64)[:MAXB], hg, We.astype(jnp.bfloat16))

    yg = pl.kernel(
        functools.partial(_gather_body, rows_per_w=ROWS_K5,
                          clamp_hi=PMAX - 1, depth=4),
        out_type=jax.ShapeDtypeStruct((T2, DP), jnp.int32),
        mesh=mesh,
        scratch_types=[
            pltpu.VMEM((ROWS_K5,), jnp.int32),
            pltpu.VMEM((4, CH, DP), jnp.int32),
            pltpu.SemaphoreType.DMA((4,)),
        ],
    )(pos, y)

    out = pl.pallas_call(
        _pairsum_kernel,
        grid=(n_blocks,),
        in_specs=[
            pl.BlockSpec((BT, 2 * DP), lambda i: (i, 0)),
            pl.BlockSpec((BT, 2), lambda i: (i, 0)),
            pl.BlockSpec((BT, 2), lambda i: (i, 0)),
            pl.BlockSpec((E, D), lambda i: (0, 0)),
        ],
        out_specs=pl.BlockSpec((BT, D), lambda i: (i, 0)),
        out_shape=jax.ShapeDtypeStruct((T, D), jnp.float32),
    )(yg.reshape(T, 2 * DP), ii, gg, be)

    return (out.reshape(B, S, D), ent[0, 0], jnp.float32(0.0))


# speedup vs baseline: 1.7032x; 1.0937x over previous
import functools

import jax
import jax.numpy as jnp
from jax import lax
from jax.experimental import pallas as pl
from jax.experimental.pallas import tpu as pltpu
from jax.experimental.pallas import tpu_sc as plsc

B, S, D, A, E = 2, 2048, 1024, 128, 8
T = B * S
T2 = 2 * T
BT = 512
BTS = 256
MAXB = T2 // BTS + E
PMAX = MAXB * BTS
NC, NS = 2, 16
NW = NC * NS
ROWS_K3 = PMAX // NW
ROWS_K5 = T2 // NW
CH = 32
DP = D // 2


def _pack(lo, hi):
    lo_b = lax.shift_right_logical(
        lax.bitcast_convert_type(lo.astype(jnp.bfloat16).astype(jnp.float32),
                                 jnp.int32), 16)
    hi_b = lax.bitcast_convert_type(
        hi.astype(jnp.bfloat16).astype(jnp.float32), jnp.int32) & (-65536)
    return hi_b | lo_b


def _unpack(pk):
    lo = lax.bitcast_convert_type(lax.shift_left(pk, 16), jnp.float32)
    hi = lax.bitcast_convert_type(pk & (-65536), jnp.float32)
    return lo, hi


def _gate_kernel(a_ref, wg_ref, bg_ref, h_ref,
                 ii_ref, gg_ref, pwe_ref, cnt_ref, bexp_ref, val_ref,
                 hp1_ref, hp2_ref, ent_ref, basev):
    i = pl.program_id(0)
    n_blocks = T // BT

    @pl.when(i == 0)
    def _():
        basev[...] = jnp.zeros((1, E), jnp.float32)
        ent_ref[0, 0] = jnp.float32(0.0)

    h = h_ref[...]
    h_bf = h.astype(jnp.bfloat16)

    wg = wg_ref[...]
    wg_h = wg[:, :D].astype(jnp.bfloat16)
    wg_a = wg[:, D:].astype(jnp.bfloat16)
    b_idx = i * BT // S
    a_bf = a_ref[pl.ds(b_idx, 1), :].astype(jnp.bfloat16)
    logits = jax.lax.dot_general(
        h_bf, wg_h, (((1,), (1,)), ((), ())),
        preferred_element_type=jnp.float32)
    logits_a = jax.lax.dot_general(
        a_bf, wg_a, (((1,), (1,)), ((), ())),
        preferred_element_type=jnp.float32)
    logits = logits + logits_a + bg_ref[...]

    m = jnp.max(logits, axis=-1, keepdims=True)
    p = jnp.exp(logits - m)
    s = jnp.sum(p, axis=-1, keepdims=True)
    g = p / s

    iota = lax.broadcasted_iota(jnp.int32, (BT, E), 1)
    m1 = jnp.max(g, axis=-1, keepdims=True)
    i1 = jnp.min(jnp.where(g == m1, iota, E), axis=-1, keepdims=True)
    gm = jnp.where(iota == i1, -jnp.inf, g)
    m2 = jnp.max(gm, axis=-1, keepdims=True)
    i2 = jnp.min(jnp.where(gm == m2, iota, E), axis=-1, keepdims=True)
    den = m1 + m2
    g1n = m1 / den
    g2n = m2 / den

    ent_ref[0, 0] += jnp.sum(g * jnp.log(g + 1e-10))

    m1hot = (iota == i1).astype(jnp.bfloat16)
    m2hot = (iota == i2).astype(jnp.bfloat16)
    r = lax.broadcasted_iota(jnp.int32, (BT, BT), 0)
    c = lax.broadcasted_iota(jnp.int32, (BT, BT), 1)
    ltri = (c < r).astype(jnp.bfloat16)
    cex = jax.lax.dot_general(
        ltri, m1hot + m2hot, (((1,), (0,)), ((), ())),
        preferred_element_type=jnp.float32)
    pvec = basev[...] + cex
    pwe1 = jnp.sum(jnp.where(iota == i1, pvec, 0.0), axis=-1, keepdims=True)
    pwe2 = jnp.sum(jnp.where(iota == i2, pvec, 0.0), axis=-1, keepdims=True)
    basev[...] += jnp.sum(m1hot + m2hot, axis=0, keepdims=True
                          ).astype(jnp.float32)

    ii_ref[...] = jnp.concatenate([i1, i2], axis=1)
    gg_ref[...] = jnp.concatenate([g1n, g2n], axis=1)
    pwe_ref[...] = jnp.concatenate([pwe1, pwe2], axis=1).astype(jnp.int32)
    hp1_ref[...] = _pack(h[:, :DP] * g1n, h[:, DP:] * g1n)
    hp2_ref[...] = _pack(h[:, :DP] * g2n, h[:, DP:] * g2n)

    @pl.when(i == n_blocks - 1)
    def _():
        ent_ref[0, 0] = ent_ref[0, 0] * jnp.float32(-1.0 / T)
        cnt = basev[...]
        lane8 = lax.broadcasted_iota(jnp.int32, (1, E), 1)
        iota64 = lax.broadcasted_iota(jnp.int32, (1, 64), 1
                                      ).astype(jnp.float32)
        bexp = jnp.full((1, 64), -1.0, jnp.float32)
        val = jnp.zeros((1, 64), jnp.float32)
        pbase = jnp.zeros((1, E), jnp.float32)
        run = jnp.float32(0.0)
        for e in range(E):
            ce = jnp.sum(jnp.where(lane8 == e, cnt, 0.0))
            cpad = jnp.floor((ce + (BTS - 1)) / BTS) * BTS
            pbase = pbase + jnp.where(lane8 == e, run, 0.0)
            blkbase = run / BTS
            bexp = bexp + (iota64 >= blkbase).astype(jnp.float32)
            val = val + jnp.where(
                bexp == e,
                jnp.clip(ce - (iota64 - blkbase) * BTS, 0.0, float(BTS)),
                0.0)
            run = run + cpad
        pbase16 = jnp.concatenate(
            [pbase, jnp.zeros((1, 8), jnp.float32)], axis=1)
        cnt_ref[...] = pbase16.astype(jnp.int32)
        bexp_ref[...] = jnp.clip(bexp, 0.0, float(E - 1)).astype(jnp.int32)
        val_ref[...] = val.astype(jnp.int32)


def _pos_kernel(ii_ref, pwe_ref, pbase_ref, sp_ref, s1_ref, s2_ref):
    lane16 = lax.broadcasted_iota(jnp.int32, (1, 16), 1)
    ii = ii_ref[...]
    acc = jnp.zeros((T, 2), jnp.int32)
    pbase = pbase_ref[...]
    for e in range(E):
        be_s = jnp.sum(jnp.where(lane16 == e, pbase, 0))
        acc = acc + jnp.where(ii == e, be_s, 0)
    sp = acc + pwe_ref[...]
    sp_ref[...] = sp
    s1_ref[...] = sp[:, 0:1]
    s2_ref[...] = sp[:, 1:2]


def _rowscatter_body(s1_hbm, s2_hbm, h1_hbm, h2_hbm, hs_hbm,
                     s1_v, s2_v, bufs1, bufs2, sem1, sem2, sem_a, sem_b):
    cid = lax.axis_index("c")
    sid = lax.axis_index("s")
    wid = sid * NC + cid
    t0 = wid * (T // NW)
    pltpu.sync_copy(s1_hbm.at[pl.ds(t0, T // NW)], s1_v)
    pltpu.sync_copy(s2_hbm.at[pl.ds(t0, T // NW)], s2_v)
    SCH = 16
    nch = (T // NW) // SCH

    def fire(c):
        return (
            pltpu.async_copy(h1_hbm.at[pl.ds(t0 + c * SCH, SCH)],
                             bufs1.at[c % 2], sem1.at[c % 2]),
            pltpu.async_copy(h2_hbm.at[pl.ds(t0 + c * SCH, SCH)],
                             bufs2.at[c % 2], sem2.at[c % 2]))

    live = fire(0)
    for c in range(nch):
        nxt = fire(c + 1) if c + 1 < nch else None
        live[0].wait()
        live[1].wait()
        sl = pl.ds(c * SCH, SCH)
        a = pltpu.async_copy(bufs1.at[c % 2], hs_hbm.at[s1_v.at[sl]], sem_a)
        b = pltpu.async_copy(bufs2.at[c % 2], hs_hbm.at[s2_v.at[sl]], sem_b)
        a.wait()
        b.wait()
        live = nxt


def _gather_body(idx_hbm, tab_hbm, out_hbm, idx_v, bufs, sem,
                 *, rows_per_w, clamp_hi, depth):
    cid = lax.axis_index("c")
    sid = lax.axis_index("s")
    wid = sid * NC + cid
    base = wid * rows_per_w
    nch = rows_per_w // CH

    pltpu.sync_copy(idx_hbm.at[pl.ds(base, rows_per_w)], idx_v)
    for k in range(rows_per_w // 16):
        sl = pl.ds(k * 16, 16)
        idx_v[sl] = jnp.clip(idx_v[sl], 0, clamp_hi)

    def fire(ch):
        return pltpu.async_copy(
            tab_hbm.at[idx_v.at[pl.ds(ch * CH, CH)]],
            bufs.at[ch % depth], sem.at[ch % depth])

    live = [fire(ch) for ch in range(min(depth, nch))]
    for ch in range(nch):
        live[ch % depth].wait()
        pltpu.sync_copy(bufs.at[ch % depth],
                        out_hbm.at[pl.ds(base + ch * CH, CH)])
        if ch + depth < nch:
            live[ch % depth] = fire(ch + depth)


def _expert_kernel(bexp_ref, hg_ref, we_ref, y_ref):
    h_lo, h_hi = _unpack(hg_ref[...])
    hf = jnp.concatenate([h_lo, h_hi], axis=1)
    hb = hf.astype(jnp.bfloat16)
    wb = we_ref[0].astype(jnp.bfloat16)
    y = jax.lax.dot_general(
        hb, wb, (((1,), (1,)), ((), ())),
        preferred_element_type=jnp.float32)
    y_ref[...] = _pack(y[:, :DP], y[:, DP:])


def _pairsum_kernel(ys_ref, ii_ref, gg_ref, be_ref, out_ref):
    ys = ys_ref[...]
    a_lo, a_hi = _unpack(ys[:, :DP])
    b_lo, b_hi = _unpack(ys[:, DP:])
    lo = a_lo + b_lo
    hi = a_hi + b_hi
    ii = ii_ref[...]
    gg = gg_ref[...]
    i1 = ii[:, 0:1]
    i2 = ii[:, 1:2]
    g1 = gg[:, 0:1]
    g2 = gg[:, 1:2]
    io8 = lax.broadcasted_iota(jnp.int32, (BT, E), 1)
    w8 = (jnp.where(io8 == i1, g1, 0.0)
          + jnp.where(io8 == i2, g2, 0.0))
    bias = jax.lax.dot_general(
        w8, be_ref[...], (((1,), (0,)), ((), ())),
        preferred_element_type=jnp.float32)
    out_ref[...] = jnp.concatenate(
        [lo + bias[:, :DP], hi + bias[:, DP:]], axis=1)


@jax.jit
def kernel(h, a, We, be, Wg, bg):
    h2 = h.reshape(T, D)
    bg2 = bg.reshape(1, E)

    n_blocks = T // BT
    ii, gg, pwe, cnt16, bexp64, val64, hp1, hp2, ent = pl.pallas_call(
        _gate_kernel,
        grid=(n_blocks,),
        in_specs=[
            pl.BlockSpec((B, A), lambda i: (0, 0)),
            pl.BlockSpec((E, D + A), lambda i: (0, 0)),
            pl.BlockSpec((1, E), lambda i: (0, 0)),
            pl.BlockSpec((BT, D), lambda i: (i, 0)),
        ],
        out_specs=[
            pl.BlockSpec((BT, 2), lambda i: (i, 0)),
            pl.BlockSpec((BT, 2), lambda i: (i, 0)),
            pl.BlockSpec((BT, 2), lambda i: (i, 0)),
            pl.BlockSpec((1, 16), lambda i: (0, 0)),
            pl.BlockSpec((1, 64), lambda i: (0, 0)),
            pl.BlockSpec((1, 64), lambda i: (0, 0)),
            pl.BlockSpec((BT, DP), lambda i: (i, 0)),
            pl.BlockSpec((BT, DP), lambda i: (i, 0)),
            pl.BlockSpec(memory_space=pltpu.SMEM),
        ],
        out_shape=[
            jax.ShapeDtypeStruct((T, 2), jnp.int32),
            jax.ShapeDtypeStruct((T, 2), jnp.float32),
            jax.ShapeDtypeStruct((T, 2), jnp.int32),
            jax.ShapeDtypeStruct((1, 16), jnp.int32),
            jax.ShapeDtypeStruct((1, 64), jnp.int32),
            jax.ShapeDtypeStruct((1, 64), jnp.int32),
            jax.ShapeDtypeStruct((T, DP), jnp.int32),
            jax.ShapeDtypeStruct((T, DP), jnp.int32),
            jax.ShapeDtypeStruct((1, 1), jnp.float32),
        ],
        scratch_shapes=[pltpu.VMEM((1, E), jnp.float32)],
        compiler_params=pltpu.CompilerParams(
            dimension_semantics=("arbitrary",)),
    )(a, Wg, bg2, h2)

    mesh = plsc.VectorSubcoreMesh(core_axis_name="c", subcore_axis_name="s")

    sp, sp1, sp2 = pl.pallas_call(
        _pos_kernel,
        in_specs=[
            pl.BlockSpec((T, 2), lambda: (0, 0)),
            pl.BlockSpec((T, 2), lambda: (0, 0)),
            pl.BlockSpec((1, 16), lambda: (0, 0)),
        ],
        out_specs=[
            pl.BlockSpec((T, 2), lambda: (0, 0)),
            pl.BlockSpec((T, 1), lambda: (0, 0)),
            pl.BlockSpec((T, 1), lambda: (0, 0)),
        ],
        out_shape=[
            jax.ShapeDtypeStruct((T, 2), jnp.int32),
            jax.ShapeDtypeStruct((T, 1), jnp.int32),
            jax.ShapeDtypeStruct((T, 1), jnp.int32),
        ],
    )(ii, pwe, cnt16)
    pos = sp.reshape(T2)

    hg = pl.kernel(
        _rowscatter_body,
        out_type=jax.ShapeDtypeStruct((PMAX, DP), jnp.int32),
        mesh=mesh,
        scratch_types=[
            pltpu.VMEM((T // NW,), jnp.int32),
            pltpu.VMEM((T // NW,), jnp.int32),
            pltpu.VMEM((2, 16, DP), jnp.int32),
            pltpu.VMEM((2, 16, DP), jnp.int32),
            pltpu.SemaphoreType.DMA((2,)),
            pltpu.SemaphoreType.DMA((2,)),
            pltpu.SemaphoreType.DMA,
            pltpu.SemaphoreType.DMA,
        ],
    )(sp1.reshape(T), sp2.reshape(T), hp1, hp2)

    y = pl.pallas_call(
        _expert_kernel,
        grid_spec=pltpu.PrefetchScalarGridSpec(
            num_scalar_prefetch=1,
            grid=(MAXB,),
            in_specs=[
                pl.BlockSpec((BTS, DP), lambda i, be_: (i, 0)),
                pl.BlockSpec((1, D, D), lambda i, be_: (be_[i], 0, 0)),
            ],
            out_specs=pl.BlockSpec((BTS, DP), lambda i, be_: (i, 0)),
        ),
        out_shape=jax.ShapeDtypeStruct((PMAX, DP), jnp.int32),
        compiler_params=pltpu.CompilerParams(
            dimension_semantics=("arbitrary",)),
    )(bexp64.reshape(64)[:MAXB], hg, We)

    yg = pl.kernel(
        functools.partial(_gather_body, rows_per_w=ROWS_K5,
                          clamp_hi=PMAX - 1, depth=4),
        out_type=jax.ShapeDtypeStruct((T2, DP), jnp.int32),
        mesh=mesh,
        scratch_types=[
            pltpu.VMEM((ROWS_K5,), jnp.int32),
            pltpu.VMEM((4, CH, DP), jnp.int32),
            pltpu.SemaphoreType.DMA((4,)),
        ],
    )(pos, y)

    out = pl.pallas_call(
        _pairsum_kernel,
        grid=(n_blocks,),
        in_specs=[
            pl.BlockSpec((BT, 2 * DP), lambda i: (i, 0)),
            pl.BlockSpec((BT, 2), lambda i: (i, 0)),
            pl.BlockSpec((BT, 2), lambda i: (i, 0)),
            pl.BlockSpec((E, D), lambda i: (0, 0)),
        ],
        out_specs=pl.BlockSpec((BT, D), lambda i: (i, 0)),
        out_shape=jax.ShapeDtypeStruct((T, D), jnp.float32),
    )(yg.reshape(T, 2 * DP), ii, gg, be)

    return (out.reshape(B, S, D), ent[0, 0], jnp.float32(0.0))
